# Initial kernel scaffold; baseline (speedup 1.0000x reference)
import jax
import jax.numpy as jnp
from jax.experimental import pallas as pl


def kernel(X, A, V, W_gg, w_ih, w_hh, b_ih, b_hh, fc1_W, fc1_b, fc2_W, fc2_b, fc3_W, fc3_b):
    raise NotImplementedError("write your pallas kernel here")


# speedup vs baseline: 4.1524x; 4.1524x over previous
import functools

import jax
import jax.numpy as jnp
from jax import lax
from jax.experimental import pallas as pl
from jax.experimental.pallas import tpu as pltpu
from jax.experimental.pallas import tpu_sc as plsc

NC = 2
NS = 16
NW = NC * NS
LANES = 16
EB = 128
BLK = 1000


@functools.lru_cache(maxsize=None)
def _make_edge_agg(n_pad, d, nb):
    mesh = plsc.VectorSubcoreMesh(core_axis_name="c", subcore_axis_name="s")
    rows_per_tile = n_pad // NS
    zc = EB
    nz = rows_per_tile // zc

    @functools.partial(
        pl.kernel,
        out_type=jax.ShapeDtypeStruct((NC, n_pad, d), jnp.float32),
        mesh=mesh,
        scratch_types=[
            pltpu.VMEM((nb, EB), jnp.int32),
            pltpu.VMEM((nb, EB), jnp.int32),
            pltpu.VMEM((nb, EB), jnp.float32),
            pltpu.VMEM((EB, d), jnp.float32),
            pltpu.VMEM_SHARED((n_pad, d), jnp.float32),
            pltpu.SemaphoreType.DMA,
        ],
    )
    def edge_agg(m_hbm, src_hbm, dst_hbm, v_hbm, out_hbm,
                 srcv, dstv, vv, rows, aggs, sem):
        c = lax.axis_index("c")
        s = lax.axis_index("s")
        w = c * NS + s
        base = s * rows_per_tile

        pltpu.sync_copy(src_hbm.at[w], srcv)
        pltpu.sync_copy(dst_hbm.at[w], dstv)
        pltpu.sync_copy(v_hbm.at[w], vv)

        zeros16 = jnp.zeros((LANES,), jnp.float32)

        def zrow(i, carry):
            for k in range(d // LANES):
                rows[i, pl.ds(k * LANES, LANES)] = zeros16
            return carry

        lax.fori_loop(0, EB, zrow, 0)
        for j in range(nz):
            pltpu.sync_copy(rows, aggs.at[pl.ds(base + j * zc, zc)])
        plsc.subcore_barrier()

        def batch(j, carry):
            pltpu.async_copy(m_hbm.at[srcv.at[j]], rows, sem).wait()

            def scale(t, inner):
                vblk = vv[j, pl.ds(t * LANES, LANES)]
                for e16 in range(LANES):
                    idxc = jnp.full((LANES,), e16, jnp.int32)
                    vsp = vblk.at[idxc].get(mode="promise_in_bounds")
                    r = t * LANES + e16
                    for k in range(d // LANES):
                        sl = pl.ds(k * LANES, LANES)
                        rows[r, sl] = rows[r, sl] * vsp
                return inner

            lax.fori_loop(0, EB // LANES, scale, 0)
            pltpu.sync_copy(rows, aggs.at[dstv.at[j]], add=True)
            return carry

        lax.fori_loop(0, nb, batch, 0)

        plsc.subcore_barrier()
        for j in range(nz):
            sl = pl.ds(base + j * zc, zc)
            pltpu.sync_copy(aggs.at[sl], rows)
            pltpu.sync_copy(rows, out_hbm.at[c, sl])

    return edge_agg


def _mm_body(x_ref, w_ref, o_ref):
    o_ref[...] = jnp.dot(x_ref[...], w_ref[...],
                         preferred_element_type=jnp.float32)


def _mm(x, w):
    n, d = x.shape
    k = w.shape[1]
    return pl.pallas_call(
        _mm_body,
        grid=(n // BLK,),
        in_specs=[pl.BlockSpec((BLK, d), lambda i: (i, 0)),
                  pl.BlockSpec((d, k), lambda i: (0, 0))],
        out_specs=pl.BlockSpec((BLK, k), lambda i: (i, 0)),
        out_shape=jax.ShapeDtypeStruct((n, k), jnp.float32),
    )(x, w)


def _gru_core(p_ref, h_ref, wih_ref, whh_ref, bih_ref, bhh_ref):
    agg = p_ref[0] + p_ref[1]
    h = h_ref[...]
    d = h.shape[1]
    gi = jnp.dot(agg, wih_ref[...], preferred_element_type=jnp.float32)
    gi = gi + bih_ref[...]
    gh = jnp.dot(h, whh_ref[...], preferred_element_type=jnp.float32)
    gh = gh + bhh_ref[...]
    r = jax.nn.sigmoid(gi[:, :d] + gh[:, :d])
    z = jax.nn.sigmoid(gi[:, d:2 * d] + gh[:, d:2 * d])
    nn = jnp.tanh(gi[:, 2 * d:] + r * gh[:, 2 * d:])
    return (1.0 - z) * nn + z * h


def _gru_mid_body(p_ref, h_ref, wih_ref, whh_ref, bih_ref, bhh_ref, wn_ref,
                  h_out, m_out):
    hn = _gru_core(p_ref, h_ref, wih_ref, whh_ref, bih_ref, bhh_ref)
    h_out[...] = hn
    m_out[...] = jnp.dot(hn, wn_ref[...], preferred_element_type=jnp.float32)


def _gru_mid(parts, h, wihT, whhT, bih2, bhh2, wnext):
    n, d = h.shape
    d3 = wihT.shape[1]
    return pl.pallas_call(
        _gru_mid_body,
        grid=(n // BLK,),
        in_specs=[pl.BlockSpec((NC, BLK, d), lambda i: (0, i, 0)),
                  pl.BlockSpec((BLK, d), lambda i: (i, 0)),
                  pl.BlockSpec((d, d3), lambda i: (0, 0)),
                  pl.BlockSpec((d, d3), lambda i: (0, 0)),
                  pl.BlockSpec((1, d3), lambda i: (0, 0)),
                  pl.BlockSpec((1, d3), lambda i: (0, 0)),
                  pl.BlockSpec((d, d), lambda i: (0, 0))],
        out_specs=[pl.BlockSpec((BLK, d), lambda i: (i, 0)),
                   pl.BlockSpec((BLK, d), lambda i: (i, 0))],
        out_shape=[jax.ShapeDtypeStruct((n, d), jnp.float32),
                   jax.ShapeDtypeStruct((n, d), jnp.float32)],
    )(parts, h, wihT, whhT, bih2, bhh2, wnext)


def _gru_fin_body(p_ref, h_ref, wih_ref, whh_ref, bih_ref, bhh_ref, s_out):
    hn = _gru_core(p_ref, h_ref, wih_ref, whh_ref, bih_ref, bhh_ref)
    s_out[...] = jnp.sum(hn, axis=0).reshape(1, 1, -1)


def _gru_fin(parts, h, wihT, whhT, bih2, bhh2):
    n, d = h.shape
    d3 = wihT.shape[1]
    return pl.pallas_call(
        _gru_fin_body,
        grid=(n // BLK,),
        in_specs=[pl.BlockSpec((NC, BLK, d), lambda i: (0, i, 0)),
                  pl.BlockSpec((BLK, d), lambda i: (i, 0)),
                  pl.BlockSpec((d, d3), lambda i: (0, 0)),
                  pl.BlockSpec((d, d3), lambda i: (0, 0)),
                  pl.BlockSpec((1, d3), lambda i: (0, 0)),
                  pl.BlockSpec((1, d3), lambda i: (0, 0))],
        out_specs=pl.BlockSpec((1, 1, d), lambda i: (i, 0, 0)),
        out_shape=jax.ShapeDtypeStruct((n // BLK, 1, d), jnp.float32),
    )(parts, h, wihT, whhT, bih2, bhh2)


def _mlp_body(inv_n, s_ref, w1_ref, b1_ref, w2_ref, b2_ref, w3_ref, b3_ref,
              o_ref):
    pooled = jnp.sum(s_ref[...], axis=0, keepdims=True) * inv_n
    x = jnp.dot(pooled, w1_ref[...], preferred_element_type=jnp.float32)
    x = jax.nn.relu(x + b1_ref[...])
    x = jnp.dot(x, w2_ref[...], preferred_element_type=jnp.float32)
    x = jax.nn.relu(x + b2_ref[...])
    x = jnp.dot(x, w3_ref[...], preferred_element_type=jnp.float32)
    o_ref[...] = x + b3_ref[...]


def _mlp(sums, n, fc1_W, fc1_b, fc2_W, fc2_b, fc3_W, fc3_b):
    return pl.pallas_call(
        functools.partial(_mlp_body, 1.0 / n),
        out_shape=jax.ShapeDtypeStruct((1, 1), jnp.float32),
    )(sums, fc1_W.T, fc1_b.reshape(1, -1), fc2_W.T, fc2_b.reshape(1, -1),
      fc3_W.T, fc3_b.reshape(1, -1))


def kernel(X, A, V, W_gg, w_ih, w_hh, b_ih, b_hh,
           fc1_W, fc1_b, fc2_W, fc2_b, fc3_W, fc3_b):
    n, d = X.shape
    e = A.shape[1]
    num_layers = W_gg.shape[0]

    nb = -(-(-(-e // NW)) // EB)
    tot = NW * nb * EB
    src = jnp.zeros((tot,), jnp.int32).at[:e].set(A[0])
    dst = jnp.zeros((tot,), jnp.int32).at[:e].set(A[1])
    vpad = jnp.zeros((tot,), jnp.float32).at[:e].set(V)
    srcp = src.reshape(NW, nb, EB)
    dstp = dst.reshape(NW, nb, EB)
    vp = vpad.reshape(NW, nb, EB)

    n_pad = -(-n // (NS * EB)) * NS * EB
    edge_agg_full = _make_edge_agg(n_pad, d, nb)

    def edge_agg(m, sp, dp, vpw):
        return edge_agg_full(m, sp, dp, vpw)[:, :n, :]

    wihT = w_ih.T
    whhT = w_hh.T
    bih2 = b_ih.reshape(1, -1)
    bhh2 = b_hh.reshape(1, -1)

    h = X
    m = _mm(h, W_gg[0])
    sums = None
    for i in range(num_layers):
        parts = edge_agg(m, srcp, dstp, vp)
        if i < num_layers - 1:
            h, m = _gru_mid(parts, h, wihT, whhT, bih2, bhh2, W_gg[i + 1])
        else:
            sums = _gru_fin(parts, h, wihT, whhT, bih2, bhh2)

    out = _mlp(sums.reshape(-1, d), n, fc1_W, fc1_b, fc2_W, fc2_b,
               fc3_W, fc3_b)
    return out.reshape(1)


# speedup vs baseline: 4.1631x; 1.0026x over previous
import functools

import jax
import jax.numpy as jnp
from jax import lax
from jax.experimental import pallas as pl
from jax.experimental.pallas import tpu as pltpu
from jax.experimental.pallas import tpu_sc as plsc

NC = 2
NS = 16
NW = NC * NS
LANES = 16
EB = 128
BLK = 1000


@functools.lru_cache(maxsize=None)
def _make_edge_agg(n_pad, d, nb):
    mesh = plsc.VectorSubcoreMesh(core_axis_name="c", subcore_axis_name="s")
    rows_per_tile = n_pad // NS
    zc = EB
    nz = rows_per_tile // zc

    @functools.partial(
        pl.kernel,
        out_type=jax.ShapeDtypeStruct((NC, n_pad, d), jnp.float32),
        mesh=mesh,
        scratch_types=[
            pltpu.VMEM((nb, EB), jnp.int32),
            pltpu.VMEM((nb, EB), jnp.int32),
            pltpu.VMEM((nb, EB), jnp.float32),
            pltpu.VMEM((EB, d), jnp.float32),
            pltpu.VMEM_SHARED((n_pad, d), jnp.float32),
            pltpu.SemaphoreType.DMA,
            pltpu.SemaphoreType.DMA,
        ],
        compiler_params=pltpu.CompilerParams(use_tc_tiling_on_sc=False),
    )
    def edge_agg(m_hbm, src_hbm, dst_hbm, v_hbm, out_hbm,
                 srcv, dstv, vv, rows, aggs, sem_g, sem_s):
        c = lax.axis_index("c")
        s = lax.axis_index("s")
        w = c * NS + s
        base = s * rows_per_tile

        pltpu.sync_copy(src_hbm.at[w], srcv)
        pltpu.sync_copy(dst_hbm.at[w], dstv)
        pltpu.sync_copy(v_hbm.at[w], vv)

        zeros16 = jnp.zeros((LANES,), jnp.float32)

        def zrow(i, carry):
            for k in range(d // LANES):
                rows[i, pl.ds(k * LANES, LANES)] = zeros16
            return carry

        lax.fori_loop(0, EB, zrow, 0)
        for j in range(nz):
            pltpu.sync_copy(rows, aggs.at[pl.ds(base + j * zc, zc)])
        plsc.subcore_barrier()

        def body(j, carry):
            pltpu.async_copy(m_hbm.at[srcv.at[j]], rows, sem_g)
            pltpu.make_async_copy(m_hbm.at[srcv.at[j]], rows, sem_g).wait()

            def scale(t, inner):
                vblk = vv[j, pl.ds(t * LANES, LANES)]
                for e16 in range(LANES):
                    idxc = jnp.full((LANES,), e16, jnp.int32)
                    vsp = vblk.at[idxc].get(mode="promise_in_bounds")
                    r = t * LANES + e16
                    for k in range(d // LANES):
                        sl = pl.ds(k * LANES, LANES)
                        rows[r, sl] = rows[r, sl] * vsp
                return inner

            lax.fori_loop(0, EB // LANES, scale, 0)

            pltpu.async_copy(rows, aggs.at[dstv.at[j]], sem_s, add=True)
            pltpu.make_async_copy(rows, aggs.at[dstv.at[j]], sem_s).wait()
            return carry

        lax.fori_loop(0, nb, body, 0)

        plsc.subcore_barrier()
        for j in range(nz):
            sl = pl.ds(base + j * zc, zc)
            pltpu.sync_copy(aggs.at[sl], rows)
            pltpu.sync_copy(rows, out_hbm.at[c, sl])

    return edge_agg


def _mm_body(x_ref, w_ref, o_ref):
    o_ref[...] = jnp.dot(x_ref[...], w_ref[...],
                         preferred_element_type=jnp.float32)


def _mm(x, w):
    n, d = x.shape
    do = w.shape[1]
    return pl.pallas_call(
        _mm_body,
        grid=(n // BLK,),
        in_specs=[pl.BlockSpec((BLK, d), lambda i: (i, 0)),
                  pl.BlockSpec((d, do), lambda i: (0, 0))],
        out_specs=pl.BlockSpec((BLK, do), lambda i: (i, 0)),
        out_shape=jax.ShapeDtypeStruct((n, do), jnp.float32),
    )(x, w)


def _gru_core(p_ref, h_ref, wih_ref, whh_ref, bih_ref, bhh_ref):
    agg = p_ref[0] + p_ref[1]
    h = h_ref[...]
    d = h.shape[1]
    gi = jnp.dot(agg, wih_ref[...], preferred_element_type=jnp.float32)
    gi = gi + bih_ref[...]
    gh = jnp.dot(h, whh_ref[...], preferred_element_type=jnp.float32)
    gh = gh + bhh_ref[...]
    r = jax.nn.sigmoid(gi[:, :d] + gh[:, :d])
    z = jax.nn.sigmoid(gi[:, d:2 * d] + gh[:, d:2 * d])
    nn = jnp.tanh(gi[:, 2 * d:] + r * gh[:, 2 * d:])
    return (1.0 - z) * nn + z * h


def _gru_mid_body(p_ref, h_ref, wih_ref, whh_ref, bih_ref, bhh_ref, wn_ref,
                  h_out, m_out):
    hn = _gru_core(p_ref, h_ref, wih_ref, whh_ref, bih_ref, bhh_ref)
    h_out[...] = hn
    m_out[...] = jnp.dot(hn, wn_ref[...], preferred_element_type=jnp.float32)


def _gru_mid(parts, h, wihT, whhT, bih2, bhh2, wnext):
    n, d = h.shape
    d3 = wihT.shape[1]
    return pl.pallas_call(
        _gru_mid_body,
        grid=(n // BLK,),
        in_specs=[pl.BlockSpec((NC, BLK, d), lambda i: (0, i, 0)),
                  pl.BlockSpec((BLK, d), lambda i: (i, 0)),
                  pl.BlockSpec((d, d3), lambda i: (0, 0)),
                  pl.BlockSpec((d, d3), lambda i: (0, 0)),
                  pl.BlockSpec((1, d3), lambda i: (0, 0)),
                  pl.BlockSpec((1, d3), lambda i: (0, 0)),
                  pl.BlockSpec((d, d), lambda i: (0, 0))],
        out_specs=[pl.BlockSpec((BLK, d), lambda i: (i, 0)),
                   pl.BlockSpec((BLK, d), lambda i: (i, 0))],
        out_shape=[jax.ShapeDtypeStruct((n, d), jnp.float32),
                   jax.ShapeDtypeStruct((n, d), jnp.float32)],
    )(parts, h, wihT, whhT, bih2, bhh2, wnext)


def _gru_fin_body(p_ref, h_ref, wih_ref, whh_ref, bih_ref, bhh_ref, s_out):
    hn = _gru_core(p_ref, h_ref, wih_ref, whh_ref, bih_ref, bhh_ref)
    s_out[...] = jnp.sum(hn, axis=0).reshape(1, 1, -1)


def _gru_fin(parts, h, wihT, whhT, bih2, bhh2):
    n, d = h.shape
    d3 = wihT.shape[1]
    return pl.pallas_call(
        _gru_fin_body,
        grid=(n // BLK,),
        in_specs=[pl.BlockSpec((NC, BLK, d), lambda i: (0, i, 0)),
                  pl.BlockSpec((BLK, d), lambda i: (i, 0)),
                  pl.BlockSpec((d, d3), lambda i: (0, 0)),
                  pl.BlockSpec((d, d3), lambda i: (0, 0)),
                  pl.BlockSpec((1, d3), lambda i: (0, 0)),
                  pl.BlockSpec((1, d3), lambda i: (0, 0))],
        out_specs=pl.BlockSpec((1, 1, d), lambda i: (i, 0, 0)),
        out_shape=jax.ShapeDtypeStruct((n // BLK, 1, d), jnp.float32),
    )(parts, h, wihT, whhT, bih2, bhh2)


def _mlp_body(inv_n, s_ref, w1_ref, b1_ref, w2_ref, b2_ref, w3_ref, b3_ref,
              o_ref):
    pooled = jnp.sum(s_ref[...], axis=0, keepdims=True) * inv_n
    x = jnp.dot(pooled, w1_ref[...], preferred_element_type=jnp.float32)
    x = jax.nn.relu(x + b1_ref[...])
    x = jnp.dot(x, w2_ref[...], preferred_element_type=jnp.float32)
    x = jax.nn.relu(x + b2_ref[...])
    x = jnp.dot(x, w3_ref[...], preferred_element_type=jnp.float32)
    o_ref[...] = x + b3_ref[...]


def _mlp(sums, n, fc1_W, fc1_b, fc2_W, fc2_b, fc3_W, fc3_b):
    return pl.pallas_call(
        functools.partial(_mlp_body, 1.0 / n),
        out_shape=jax.ShapeDtypeStruct((1, 1), jnp.float32),
    )(sums, fc1_W.T, fc1_b.reshape(1, -1), fc2_W.T, fc2_b.reshape(1, -1),
      fc3_W.T, fc3_b.reshape(1, -1))


def kernel(X, A, V, W_gg, w_ih, w_hh, b_ih, b_hh,
           fc1_W, fc1_b, fc2_W, fc2_b, fc3_W, fc3_b):
    n, d = X.shape
    e = A.shape[1]
    num_layers = W_gg.shape[0]

    nb = -(-e // (NW * EB))
    tot = NW * nb * EB

    def part(flat):
        return flat.reshape(NW, nb, EB)

    srcp = part(jnp.zeros((tot,), jnp.int32).at[:e].set(A[0]))
    dstp = part(jnp.zeros((tot,), jnp.int32).at[:e].set(A[1]))
    vp = part(jnp.zeros((tot,), jnp.float32).at[:e].set(V))

    n_pad = -(-n // (NS * EB)) * NS * EB
    edge_agg = _make_edge_agg(n_pad, d, nb)

    wihT = w_ih.T
    whhT = w_hh.T
    bih2 = b_ih.reshape(1, -1)
    bhh2 = b_hh.reshape(1, -1)

    h = X
    m = _mm(h, W_gg[0])
    sums = None
    for i in range(num_layers):
        parts = edge_agg(m, srcp, dstp, vp)
        if i < num_layers - 1:
            h, m = _gru_mid(parts, h, wihT, whhT, bih2, bhh2, W_gg[i + 1])
        else:
            sums = _gru_fin(parts, h, wihT, whhT, bih2, bhh2)

    out = _mlp(sums.reshape(-1, d), n, fc1_W, fc1_b, fc2_W, fc2_b,
               fc3_W, fc3_b)
    return out.reshape(1)
